# Initial kernel scaffold; baseline (speedup 1.0000x reference)
import jax
import jax.numpy as jnp
from jax.experimental import pallas as pl


def kernel(logits, labels, features, sample_num_per_cls):
    raise NotImplementedError("write your pallas kernel here")


# speedup vs baseline: 7.6212x; 7.6212x over previous
import jax
import jax.numpy as jnp
from jax.experimental import pallas as pl

_K = 100
_KP = 128
_F_SCORE = 1.2447
_BETA = 0.999


def _homovar_kernel(logits_ref, labels_ref, features_ref, counts_ref, out_ref):
    f = features_ref[:]
    lab = labels_ref[:]
    counts = counts_ref[:]
    B = f.shape[0]

    citer = jax.lax.broadcasted_iota(jnp.int32, (1, _KP), 1)
    valid = (citer < _K).astype(jnp.float32)
    oh = (lab == jax.lax.broadcasted_iota(jnp.int32, (B, _KP), 1)).astype(
        jnp.float32)

    segsum = jnp.dot(oh.T, f, preferred_element_type=jnp.float32)
    classmean = segsum / counts.T
    m = jnp.dot(oh, classmean, preferred_element_type=jnp.float32)

    z = jnp.sum(jnp.abs(f - m) * (f != 0.0).astype(jnp.float32),
                axis=1, keepdims=True)

    s = jnp.sum(oh * z, axis=0, keepdims=True)
    zi_mean = s / counts
    z_mean = jnp.sum(zi_mean * valid) / _K
    n_total = jnp.sum(counts * valid)

    zi_g = jnp.sum(oh * zi_mean, axis=1, keepdims=True)
    ssw = jnp.sum((z - zi_g) ** 2 *
                  (z != 0.0).astype(jnp.float32)) / (n_total - _K)
    sb = (zi_mean - z_mean) ** 2 * counts
    ssb = jnp.sum(sb * valid) / (_K - 1)

    cq = _F_SCORE * ssw * (_K - 1) - (ssb * (_K - 1) - sb)
    a = z_mean ** 2
    b = -(2.0 * z_mean * s + cq)
    cc = s ** 2
    disc = jnp.sqrt(b * b - 4.0 * a * cc)
    n_lb = jnp.abs((-b - disc) / (2.0 * a))
    n_ub = jnp.abs((-b + disc) / (2.0 * a))

    beta = jnp.where(
        counts < n_lb,
        jnp.power(_BETA, 1.0 / (n_lb - counts)),
        jnp.where(counts > n_ub,
                  jnp.power(_BETA, 1.0 / (counts - n_ub)),
                  _BETA))
    eff = 1.0 - jnp.power(beta, counts)
    w_cls = (1.0 - beta) / eff
    w_cls = jnp.where(valid > 0.0, w_cls, 0.0)
    w_cls = w_cls / jnp.sum(w_cls) * _K
    w_n = jnp.sum(oh * w_cls, axis=1, keepdims=True)

    lg = logits_ref[:]
    mx = jnp.max(lg, axis=1, keepdims=True)
    e = jnp.exp(lg - mx)
    pred = e / jnp.sum(e, axis=1, keepdims=True)
    log_p = jnp.maximum(jnp.log(pred), -100.0)
    log_1mp = jnp.maximum(jnp.log(1.0 - pred), -100.0)
    bce = -(oh * log_p + (1.0 - oh) * log_1mp)
    total = jnp.sum(w_n * bce * valid, axis=None, keepdims=True)
    out_ref[:, :] = total / (B * _K)


def kernel(logits, labels, features, sample_num_per_cls):
    bsz, k = logits.shape
    lg = jnp.full((bsz, _KP), -1e9, dtype=jnp.float32).at[:, :k].set(logits)
    counts = jnp.ones((1, _KP), dtype=jnp.float32).at[0, :k].set(
        sample_num_per_cls)
    lab = labels.reshape(bsz, 1).astype(jnp.int32)
    out = pl.pallas_call(
        _homovar_kernel,
        out_shape=jax.ShapeDtypeStruct((1, 1), jnp.float32),
    )(lg, lab, features, counts)
    return out[0, 0]


# speedup vs baseline: 12.3053x; 1.6146x over previous
import jax
import jax.numpy as jnp
from jax.experimental import pallas as pl

_F_SCORE = 1.2447
_BETA = 0.999


def _homovar_kernel(logits_ref, labels_ref, features_ref, counts_ref, out_ref):
    f = features_ref[:]
    lab_row = labels_ref[:].reshape(1, -1)
    counts_col = counts_ref[:].reshape(-1, 1)
    b_sz = f.shape[0]
    k = counts_col.shape[0]

    lab_col = lab_row.T
    oh = (lab_col == jax.lax.broadcasted_iota(jnp.int32, (b_sz, k), 1)
          ).astype(jnp.float32)
    oht = (lab_row == jax.lax.broadcasted_iota(jnp.int32, (k, b_sz), 0)
           ).astype(jnp.float32)

    inv_counts = 1.0 / counts_col
    segsum = jnp.dot(oht, f, preferred_element_type=jnp.float32)
    g = jnp.dot(oh, segsum, preferred_element_type=jnp.float32)
    invc_n = jnp.dot(oh, inv_counts, preferred_element_type=jnp.float32)
    m = g * invc_n

    z = jnp.sum(jnp.abs(f - m) * (f != 0.0).astype(jnp.float32),
                axis=1, keepdims=True)

    s = jnp.dot(oht, z, preferred_element_type=jnp.float32)
    zi_mean = s * inv_counts
    z_mean = jnp.sum(zi_mean) / k
    n_total = jnp.sum(counts_col)

    zi_g = jnp.dot(oh, zi_mean, preferred_element_type=jnp.float32)
    ssw = jnp.sum((z - zi_g) ** 2 *
                  (z != 0.0).astype(jnp.float32)) / (n_total - k)
    sb = (zi_mean - z_mean) ** 2 * counts_col
    ssb = jnp.sum(sb) / (k - 1)

    cq = _F_SCORE * ssw * (k - 1) - (ssb * (k - 1) - sb)
    a = z_mean ** 2
    b = -(2.0 * z_mean * s + cq)
    cc = s ** 2
    disc = jnp.sqrt(b * b - 4.0 * a * cc)
    n_lb = jnp.abs((-b - disc) / (2.0 * a))
    n_ub = jnp.abs((-b + disc) / (2.0 * a))

    beta = jnp.where(
        counts_col < n_lb,
        jnp.power(_BETA, 1.0 / (n_lb - counts_col)),
        jnp.where(counts_col > n_ub,
                  jnp.power(_BETA, 1.0 / (counts_col - n_ub)),
                  _BETA))
    eff = 1.0 - jnp.power(beta, counts_col)
    w_cls = (1.0 - beta) / eff
    w_cls = w_cls / jnp.sum(w_cls) * k
    w_n = jnp.dot(oh, w_cls, preferred_element_type=jnp.float32)

    lg = logits_ref[:]
    mx = jnp.max(lg, axis=1, keepdims=True)
    e = jnp.exp(lg - mx)
    pred = e / jnp.sum(e, axis=1, keepdims=True)
    log_p = jnp.maximum(jnp.log(pred), -100.0)
    log_1mp = jnp.maximum(jnp.log(1.0 - pred), -100.0)
    bce = -(oh * log_p + (1.0 - oh) * log_1mp)
    total = jnp.sum(w_n * bce, axis=None, keepdims=True)
    out_ref[:, :] = total / (b_sz * k)


def kernel(logits, labels, features, sample_num_per_cls):
    out = pl.pallas_call(
        _homovar_kernel,
        out_shape=jax.ShapeDtypeStruct((1, 1), jnp.float32),
    )(logits, labels.astype(jnp.int32), features, sample_num_per_cls)
    return out[0, 0]
